# Initial kernel scaffold; baseline (speedup 1.0000x reference)
import jax
import jax.numpy as jnp
from jax.experimental import pallas as pl


def kernel(words, table):
    raise NotImplementedError("write your pallas kernel here")


# speedup vs baseline: 5.7453x; 5.7453x over previous
import functools

import jax
import jax.numpy as jnp
from jax import lax
from jax.experimental import pallas as pl
from jax.experimental.pallas import tpu as pltpu
from jax.experimental.pallas import tpu_sc as plsc

BATCH = 1024
SEQ = 200
EMBED = 128

NUM_CORES = 2
NUM_SUBCORES = 16
NW = NUM_CORES * NUM_SUBCORES
N_TOTAL = BATCH * SEQ
PER_W = N_TOTAL // NW
CHUNK = 128
NCHUNK = PER_W // CHUNK

_mesh = plsc.VectorSubcoreMesh(core_axis_name="c", subcore_axis_name="s")


@functools.partial(
    pl.kernel,
    mesh=_mesh,
    out_type=jax.ShapeDtypeStruct((N_TOTAL, EMBED), jnp.float32),
    scratch_types=[
        pltpu.VMEM((NCHUNK, CHUNK), jnp.int32),
        pltpu.VMEM((CHUNK, EMBED), jnp.float32),
        pltpu.SemaphoreType.DMA,
    ],
)
def _embed_sc(words_hbm, table_hbm, out_hbm, idx_v, rows_v, sem):
    wid = lax.axis_index("s") * NUM_CORES + lax.axis_index("c")
    base = wid * PER_W
    pltpu.sync_copy(words_hbm.at[wid], idx_v)

    def body(j, carry):
        pltpu.async_copy(table_hbm.at[idx_v.at[j]], rows_v, sem).wait()
        pltpu.sync_copy(rows_v, out_hbm.at[pl.ds(base + j * CHUNK, CHUNK)])
        return carry

    lax.fori_loop(0, NCHUNK, body, 0)


def kernel(words, table):
    words_r = words.reshape(NW, NCHUNK, CHUNK)
    out = _embed_sc(words_r, table)
    return out.reshape(BATCH, SEQ, EMBED)


# speedup vs baseline: 7.3056x; 1.2716x over previous
import functools

import jax
import jax.numpy as jnp
from jax import lax
from jax.experimental import pallas as pl
from jax.experimental.pallas import tpu as pltpu
from jax.experimental.pallas import tpu_sc as plsc

BATCH = 1024
SEQ = 200
EMBED = 128

NUM_CORES = 2
NUM_SUBCORES = 16
NW = NUM_CORES * NUM_SUBCORES
N_TOTAL = BATCH * SEQ
PER_W = N_TOTAL // NW
CHUNK = 128
NCHUNK = PER_W // CHUNK
NBUF = 2
NITER = NCHUNK // NBUF

_mesh = plsc.VectorSubcoreMesh(core_axis_name="c", subcore_axis_name="s")


@functools.partial(
    pl.kernel,
    mesh=_mesh,
    out_type=jax.ShapeDtypeStruct((N_TOTAL, EMBED), jnp.float32),
    scratch_types=[
        pltpu.VMEM((NCHUNK, CHUNK), jnp.int32),
        *[pltpu.VMEM((CHUNK, EMBED), jnp.float32) for _ in range(NBUF)],
        *[pltpu.SemaphoreType.DMA for _ in range(2 * NBUF)],
    ],
)
def _embed_sc(words_hbm, table_hbm, out_hbm, idx_v, *bufs_and_sems):
    rows = bufs_and_sems[:NBUF]
    gsem = bufs_and_sems[NBUF:2 * NBUF]
    ssem = bufs_and_sems[2 * NBUF:]

    wid = lax.axis_index("s") * NUM_CORES + lax.axis_index("c")
    base = wid * PER_W
    pltpu.sync_copy(words_hbm.at[wid], idx_v)

    def fire_gather(j, b):
        pltpu.async_copy(table_hbm.at[idx_v.at[j]], rows[b], gsem[b])

    def wait_gather(b):
        pltpu.make_async_copy(table_hbm.at[idx_v.at[0]], rows[b], gsem[b]).wait()

    def fire_scatter(j, b):
        pltpu.async_copy(rows[b], out_hbm.at[pl.ds(base + j * CHUNK, CHUNK)],
                         ssem[b])

    def wait_scatter(b):
        pltpu.make_async_copy(rows[b], out_hbm.at[pl.ds(base, CHUNK)],
                              ssem[b]).wait()

    for b in range(NBUF):
        fire_gather(b, b)

    def body(i, carry):
        j0 = i * NBUF
        for b in range(NBUF):
            wait_gather(b)
            fire_scatter(j0 + b, b)
        for b in range(NBUF):
            @pl.when(i + 1 < NITER)
            def _():
                wait_scatter(b)
                fire_gather(j0 + NBUF + b, b)
        return carry

    lax.fori_loop(0, NITER, body, 0)

    for b in range(NBUF):
        wait_scatter(b)


def kernel(words, table):
    words_r = words.reshape(NW, NCHUNK, CHUNK)
    out = _embed_sc(words_r, table)
    return out.reshape(BATCH, SEQ, EMBED)


# speedup vs baseline: 7.7518x; 1.0611x over previous
import functools

import jax
import jax.numpy as jnp
from jax import lax
from jax.experimental import pallas as pl
from jax.experimental.pallas import tpu as pltpu
from jax.experimental.pallas import tpu_sc as plsc

BATCH = 1024
SEQ = 200
EMBED = 128

NUM_CORES = 2
NUM_SUBCORES = 16
NW = NUM_CORES * NUM_SUBCORES
N_TOTAL = BATCH * SEQ
PER_W = N_TOTAL // NW
CHUNK = 128
NCHUNK = PER_W // CHUNK
NBUF = 5
NITER = NCHUNK // NBUF

_mesh = plsc.VectorSubcoreMesh(core_axis_name="c", subcore_axis_name="s")


@functools.partial(
    pl.kernel,
    mesh=_mesh,
    out_type=jax.ShapeDtypeStruct((N_TOTAL, EMBED), jnp.float32),
    scratch_types=[
        pltpu.VMEM((NCHUNK, CHUNK), jnp.int32),
        *[pltpu.VMEM((CHUNK, EMBED), jnp.float32) for _ in range(NBUF)],
        *[pltpu.SemaphoreType.DMA for _ in range(2 * NBUF)],
    ],
)
def _embed_sc(words_hbm, table_hbm, out_hbm, idx_v, *bufs_and_sems):
    rows = bufs_and_sems[:NBUF]
    gsem = bufs_and_sems[NBUF:2 * NBUF]
    ssem = bufs_and_sems[2 * NBUF:]

    wid = lax.axis_index("s") * NUM_CORES + lax.axis_index("c")
    base = wid * PER_W
    pltpu.sync_copy(words_hbm.at[wid], idx_v)

    def fire_gather(j, b):
        pltpu.async_copy(table_hbm.at[idx_v.at[j]], rows[b], gsem[b])

    def wait_gather(b):
        pltpu.make_async_copy(table_hbm.at[idx_v.at[0]], rows[b], gsem[b]).wait()

    def fire_scatter(j, b):
        pltpu.async_copy(rows[b], out_hbm.at[pl.ds(base + j * CHUNK, CHUNK)],
                         ssem[b])

    def wait_scatter(b):
        pltpu.make_async_copy(rows[b], out_hbm.at[pl.ds(base, CHUNK)],
                              ssem[b]).wait()

    for b in range(NBUF):
        fire_gather(b, b)

    def body(i, carry):
        j0 = i * NBUF
        for b in range(NBUF):
            wait_gather(b)
            fire_scatter(j0 + b, b)
        for b in range(NBUF):
            @pl.when(i + 1 < NITER)
            def _():
                wait_scatter(b)
                fire_gather(j0 + NBUF + b, b)
        return carry

    lax.fori_loop(0, NITER, body, 0)

    for b in range(NBUF):
        wait_scatter(b)


def kernel(words, table):
    words_r = words.reshape(NW, NCHUNK, CHUNK)
    out = _embed_sc(words_r, table)
    return out.reshape(BATCH, SEQ, EMBED)
